# baseline (speedup 1.0000x reference)
import functools

import jax
import jax.numpy as jnp
from jax.experimental import pallas as pl
from jax.experimental.pallas import tpu as pltpu
from jax.experimental.pallas import tpu_sc as plsc

_B, _S, _H = 4, 8192, 2048


def _entity_gather(pos1_hbm, pos2_hbm, table_hbm, out1_hbm, out2_hbm,
                   p1_smem, p2_smem, sem):
    c1 = pltpu.async_copy(pos1_hbm, p1_smem, sem)
    c2 = pltpu.async_copy(pos2_hbm, p2_smem, sem)
    c1.wait()
    c2.wait()
    for b in range(_B):
        r1 = p1_smem[b, 0] + b * _S
        pltpu.async_copy(table_hbm.at[pl.ds(r1, 1)], out1_hbm.at[pl.ds(b, 1)], sem)
        r2 = p2_smem[b, 0] + b * _S
        pltpu.async_copy(table_hbm.at[pl.ds(r2, 1)], out2_hbm.at[pl.ds(b, 1)], sem)
    for b in range(_B):
        pltpu.make_async_copy(
            table_hbm.at[pl.ds(0, 1)], out1_hbm.at[pl.ds(b, 1)], sem
        ).wait()
        pltpu.make_async_copy(
            table_hbm.at[pl.ds(0, 1)], out2_hbm.at[pl.ds(b, 1)], sem
        ).wait()


@jax.jit
def _run(table, pos1, pos2):
    mesh = plsc.ScalarSubcoreMesh(axis_name="c", num_cores=1)
    f = functools.partial(
        pl.kernel,
        mesh=mesh,
        out_type=(
            jax.ShapeDtypeStruct((_B, _H), jnp.float32),
            jax.ShapeDtypeStruct((_B, _H), jnp.float32),
        ),
        scratch_types=[
            pltpu.SMEM((_B, 2), jnp.int32),
            pltpu.SMEM((_B, 2), jnp.int32),
            pltpu.SemaphoreType.DMA,
        ],
    )(_entity_gather)
    return f(pos1, pos2, table)


def kernel(token_embs, pos1, pos2, mask):
    B, S, H = token_embs.shape
    table = token_embs.reshape(B * S, H)
    return _run(table, pos1.astype(jnp.int32), pos2.astype(jnp.int32))


# speedup vs baseline: 1.0157x; 1.0157x over previous
import functools

import jax
import jax.numpy as jnp
from jax.experimental import pallas as pl
from jax.experimental.pallas import tpu as pltpu
from jax.experimental.pallas import tpu_sc as plsc

_B, _S, _H = 4, 8192, 2048


def _entity_gather(pos_hbm, table_hbm, out1_hbm, out2_hbm, pos_smem, sem):
    pltpu.sync_copy(pos_hbm, pos_smem)
    for b in range(_B):
        r1 = pos_smem[b] + b * _S
        pltpu.async_copy(table_hbm.at[pl.ds(r1, 1)], out1_hbm.at[pl.ds(b, 1)], sem)
        r2 = pos_smem[_B + b] + b * _S
        pltpu.async_copy(table_hbm.at[pl.ds(r2, 1)], out2_hbm.at[pl.ds(b, 1)], sem)
    for b in range(_B):
        pltpu.make_async_copy(
            table_hbm.at[pl.ds(0, 1)], out1_hbm.at[pl.ds(b, 1)], sem
        ).wait()
        pltpu.make_async_copy(
            table_hbm.at[pl.ds(0, 1)], out2_hbm.at[pl.ds(b, 1)], sem
        ).wait()


@jax.jit
def _run(table, posflat):
    mesh = plsc.ScalarSubcoreMesh(axis_name="c", num_cores=1)
    f = functools.partial(
        pl.kernel,
        mesh=mesh,
        out_type=(
            jax.ShapeDtypeStruct((_B, _H), jnp.float32),
            jax.ShapeDtypeStruct((_B, _H), jnp.float32),
        ),
        scratch_types=[
            pltpu.SMEM((2 * _B,), jnp.int32),
            pltpu.SemaphoreType.DMA,
        ],
    )(_entity_gather)
    return f(posflat, table)


def kernel(token_embs, pos1, pos2, mask):
    B, S, H = token_embs.shape
    table = token_embs.reshape(B * S, H)
    posflat = jnp.concatenate([pos1[:, 0], pos2[:, 0]]).astype(jnp.int32)
    return _run(table, posflat)
